# baseline (speedup 1.0000x reference)
import functools
import numpy as np
import jax
import jax.numpy as jnp
from jax import lax
from jax.experimental import pallas as pl
from jax.experimental.pallas import tpu as pltpu
from jax.experimental.pallas import tpu_sc as plsc

_OUT_H = 7
_OUT_W = 7
_SCALE = 0.125
_G = 2
_O = 8
_NBIN = _OUT_H * _OUT_W
_NCHUNK = 7
_COLS = _NCHUNK * 128
_PCOL = 880
_RBLK = 64


def _const_table():
    cols = np.arange(_COLS)
    bin_ = cols >> 4
    s = (cols >> 2) & 3
    k = cols & 3
    h = np.minimum(bin_ // _OUT_W, _OUT_H - 1)
    w = bin_ % _OUT_W
    sh = s >> 1
    sw = s & 1
    t = np.zeros((8, _COLS), np.float32)
    t[0] = h
    t[1] = w
    t[2] = (sh + 0.5) / _G
    t[3] = (sw + 0.5) / _G
    t[4] = (k < 2)
    t[5] = (k % 2 == 0)
    t[6] = (bin_ < _NBIN)
    return jnp.asarray(t)


def _prep_body(rois_ref, tab_ref, idx_ref, w_ref, scal_ref, *, H, W):
    r = rois_ref[...]
    b = r[:, 0:1]
    cx = r[:, 1:2] * _SCALE
    cy = r[:, 2:3] * _SCALE
    rw = jnp.maximum(r[:, 3:4] * _SCALE, 1.0)
    rh = jnp.maximum(r[:, 4:5] * _SCALE, 1.0)
    th = r[:, 5:6]
    cos_t = jnp.cos(th)
    sin_t = jnp.sin(th)
    binh = rh / _OUT_H
    binw = rw / _OUT_W
    bh = tab_ref[0:1, :]
    bw = tab_ref[1:2, :]
    sy = tab_ref[2:3, :]
    sx = tab_ref[3:4, :]
    ysel = tab_ref[4:5, :]
    xsel = tab_ref[5:6, :]
    wmask = tab_ref[6:7, :]
    yy = rh * (-0.5) + (bh + sy) * binh
    xx = rw * (-0.5) + (bw + sx) * binw
    y = yy * cos_t - xx * sin_t + cy
    x = yy * sin_t + xx * cos_t + cx
    Hf = float(H)
    Wf = float(W)
    valid = ((y >= -1.0) & (y <= Hf) & (x >= -1.0) & (x <= Wf)).astype(jnp.float32)
    yc = jnp.maximum(y, 0.0)
    yl0 = jnp.floor(yc)
    condy = yl0 >= Hf - 1.0
    y_low = jnp.where(condy, Hf - 1.0, yl0)
    y_high = jnp.where(condy, Hf - 1.0, jnp.minimum(yl0 + 1.0, Hf - 1.0))
    yc = jnp.where(condy, Hf - 1.0, yc)
    ly = yc - y_low
    hy = 1.0 - ly
    xc = jnp.maximum(x, 0.0)
    xl0 = jnp.floor(xc)
    condx = xl0 >= Wf - 1.0
    x_low = jnp.where(condx, Wf - 1.0, xl0)
    x_high = jnp.where(condx, Wf - 1.0, jnp.minimum(xl0 + 1.0, Wf - 1.0))
    xc = jnp.where(condx, Wf - 1.0, xc)
    lx = xc - x_low
    hx = 1.0 - lx
    wy = jnp.where(ysel > 0.0, hy, ly)
    wx = jnp.where(xsel > 0.0, hx, lx)
    wgt = wy * wx * valid * (0.25 * wmask)
    idxf = b * (Hf * Wf) + y_low * Wf + x_low
    indf = th * (_O / (2.0 * np.pi))
    indfl = jnp.floor(indf)
    l_var = indf - indfl
    r_var = 1.0 - l_var
    ind_i = indfl - 8.0 * jnp.floor(indfl * 0.125)
    colid = lax.broadcasted_iota(jnp.int32, wgt.shape, 1)
    w_out = jnp.where(colid == _PCOL, r_var,
                      jnp.where(colid == _PCOL + 1, l_var, wgt))
    idx_out = jnp.where(colid < _NBIN * 16, idxf,
                        jnp.where(colid == _PCOL, ind_i, 0.0))
    idx_ref[...] = idx_out.astype(jnp.int32)
    w_ref[...] = w_out
    yl_m = jnp.where(wmask > 0.0, y_low, Hf)
    xl_m = jnp.where(wmask > 0.0, x_low, Wf)
    y0 = jnp.minimum(jnp.min(yl_m, axis=1, keepdims=True), Hf - 40.0)
    x0 = jnp.minimum(jnp.floor(jnp.min(xl_m, axis=1, keepdims=True) * 0.125) * 8.0,
                     Wf - 48.0)
    c2 = lax.broadcasted_iota(jnp.int32, scal_ref.shape, 1)
    scal = jnp.where(c2 == 0, b,
           jnp.where(c2 == 1, y0,
           jnp.where(c2 == 2, x0,
           jnp.where(c2 == 3, ind_i,
           jnp.where(c2 == 4, r_var,
           jnp.where(c2 == 5, l_var, 0.0))))))
    scal_ref[...] = scal


def _sc_body(feats_hbm, idx_hbm, w_hbm, out_hbm,
             idx_v, w_v, rows_v, pooled_v, out_v, semA, semB,
             *, rois_per_tile):
    cid = lax.axis_index("c")
    sid = lax.axis_index("s")
    wid = sid * 2 + cid

    def start(c, buf, sem):
        pltpu.make_async_copy(feats_hbm.at[idx_v.at[c]], rows_v.at[buf], sem).start()

    def wait(buf, sem):
        pltpu.make_async_copy(feats_hbm.at[idx_v.at[0]], rows_v.at[buf], sem).wait()

    def compute(c, buf):
        def lb_body(lb, _):
            base = lb * 16
            wvec = w_v[c, pl.ds(base, 16)]
            ws = [wvec[k] for k in range(16)]
            binrow = (c * 8 + lb) * 256
            for j2 in range(8):
                accA = None
                accB = None
                for s in range(4):
                    for seg in range(4):
                        wv = rows_v[buf, lb * 4 + s, pl.ds(seg * 128 + 16 * j2, 16)]
                        a = plsc.bitcast(lax.shift_left(wv, 16), jnp.float32)
                        bo = plsc.bitcast(wv & jnp.int32(-65536), jnp.float32)
                        wk = ws[s * 4 + seg]
                        if accA is None:
                            accA = wk * a
                            accB = wk * bo
                        else:
                            accA = accA + wk * a
                            accB = accB + wk * bo
                pooled_v[pl.ds(binrow + 32 * j2, 16)] = accA
                pooled_v[pl.ds(binrow + 32 * j2 + 16, 16)] = accB
            return 0
        lax.fori_loop(0, 8, lb_body, 0)

    def roi_body(i, _):
        roi = wid * rois_per_tile + i
        pltpu.sync_copy(idx_hbm.at[roi], idx_v)
        pltpu.sync_copy(w_hbm.at[roi], w_v)
        pvec_i = idx_v[6, pl.ds(16, 16)]
        pvec_w = w_v[6, pl.ds(112, 16)]
        ind = pvec_i[12]
        rv = pvec_w[0]
        lv = pvec_w[1]
        start(0, 0, semA)

        def pair_body(t, _):
            c0 = 2 * t
            start(c0 + 1, 1, semB)
            wait(0, semA)
            compute(c0, 0)
            start(c0 + 2, 0, semA)
            wait(1, semB)
            compute(c0 + 1, 1)
            return 0
        lax.fori_loop(0, 3, pair_body, 0)
        wait(0, semA)
        compute(6, 0)

        iota = lax.iota(jnp.int32, 16)
        for j in range(16):
            cvec = iota + 16 * j
            grp = cvec & (-8)
            o = cvec & 7
            sA = grp | ((o - ind) & 7)
            sB = grp | ((o - ind + 1) & 7)
            pA = (sA & (-32)) | ((sA & 1) << 4) | ((sA & 31) >> 1)
            pB = (sB & (-32)) | ((sB & 1) << 4) | ((sB & 31) >> 1)
            dstb = cvec * _NBIN

            def blend_body(bn, _):
                a = plsc.load_gather(pooled_v, [pA + bn * 256])
                bb = plsc.load_gather(pooled_v, [pB + bn * 256])
                plsc.store_scatter(out_v, [dstb + bn], rv * a + lv * bb)
                return 0
            lax.fori_loop(0, _NBIN, blend_body, 0)
        pltpu.sync_copy(out_v, out_hbm.at[roi])
        return 0
    lax.fori_loop(0, rois_per_tile, roi_body, 0)


def _tc_roi_body(scal_sm, idxT_ref, wT_ref, feats_any, out_ref,
                 patch_v, semP0, semP1, *, H, W, n_tc):
    i = pl.program_id(0)
    HW = H * W

    def getp(r):
        return (scal_sm[r, 0].astype(jnp.int32),
                scal_sm[r, 1].astype(jnp.int32),
                pl.multiple_of(scal_sm[r, 2].astype(jnp.int32), 8))

    def dma(r, buf, sem):
        b, y0, x0 = getp(r)
        return pltpu.make_async_copy(
            feats_any.at[b, pl.ds(y0, 40), pl.ds(x0, 48), :],
            patch_v.at[buf, pl.ds(0, 40)], sem)

    @pl.when(i == 0)
    def _first():
        patch_v[0, 40:42] = jnp.zeros((2, 48, 256), jnp.float32)
        patch_v[1, 40:42] = jnp.zeros((2, 48, 256), jnp.float32)
        dma(0, 0, semP0).start()

    nxt = jnp.minimum(i + 1, n_tc - 1)

    @pl.when((i + 1 < n_tc) & (i % 2 == 0))
    def _s1():
        dma(nxt, 1, semP1).start()

    @pl.when((i + 1 < n_tc) & (i % 2 == 1))
    def _s0():
        dma(nxt, 0, semP0).start()

    @pl.when(i % 2 == 0)
    def _w0():
        dma(i, 0, semP0).wait()

    @pl.when(i % 2 == 1)
    def _w1():
        dma(i, 1, semP1).wait()

    buf = i % 2
    b, y0, x0 = getp(i)
    pflat = patch_v[buf].reshape(42 * 48, 256)
    pbf = pflat.astype(jnp.bfloat16)
    idxv = idxT_ref[0]
    rel = idxv - b * HW
    yq = lax.shift_right_arithmetic(rel, 7)
    xq = rel & (W - 1)
    cell = (yq - y0) * 48 + (xq - x0)
    ci = lax.broadcasted_iota(jnp.int32, (224, 1920), 1)
    P1 = jnp.where(ci == cell, 1.0, 0.0).astype(jnp.bfloat16)
    res = None
    for k, off in enumerate((0, 1, 48, 49)):
        rk = lax.dot_general(P1, pbf[off:off + 1920, :],
                             (((1,), (0,)), ((), ())),
                             preferred_element_type=jnp.float32)
        wk = wT_ref[0, k]
        res = wk * rk if res is None else res + wk * rk
    bins = res.reshape(56, 4, 256).sum(axis=1)[:49]
    ind = scal_sm[i, 3].astype(jnp.int32)
    rv = scal_sm[i, 4]
    lv = scal_sm[i, 5]
    cp = lax.broadcasted_iota(jnp.int32, (256, 256), 0)
    cc = lax.broadcasted_iota(jnp.int32, (256, 256), 1)
    srcA = (cc & (-8)) | (((cc & 7) - ind) & 7)
    srcB = (cc & (-8)) | (((cc & 7) - ind + 1) & 7)
    B = (jnp.where(cp == srcA, rv, 0.0) + jnp.where(cp == srcB, lv, 0.0))
    outv = lax.dot_general(bins, B, (((1,), (0,)), ((), ())),
                           preferred_element_type=jnp.float32)
    out_ref[0] = outv


def kernel(features, rois):
    N, C, H, W = features.shape
    R = rois.shape[0]
    NHW = N * H * W
    feats_nhwc = jnp.transpose(features, (0, 2, 3, 1))
    fb = feats_nhwc.reshape(NHW, C).astype(jnp.bfloat16)
    fb = jnp.pad(fb, ((0, W + 2), (0, 0)))
    quad = jnp.concatenate(
        [fb[0:NHW], fb[1:NHW + 1], fb[W:NHW + W], fb[W + 1:NHW + W + 1]], axis=1)
    feats = lax.bitcast_convert_type(quad.reshape(NHW, 2 * C, 2), jnp.int32)
    rois_p = jnp.pad(rois, ((0, 0), (0, 128 - rois.shape[1])))
    tab = _const_table()
    idx_all, w_all, scal_all = pl.pallas_call(
        functools.partial(_prep_body, H=H, W=W),
        grid=(R // _RBLK,),
        in_specs=[
            pl.BlockSpec((_RBLK, 128), lambda i: (i, 0)),
            pl.BlockSpec((8, _COLS), lambda i: (0, 0)),
        ],
        out_specs=[
            pl.BlockSpec((_RBLK, _COLS), lambda i: (i, 0)),
            pl.BlockSpec((_RBLK, _COLS), lambda i: (i, 0)),
            pl.BlockSpec((_RBLK, 128), lambda i: (i, 0)),
        ],
        out_shape=[
            jax.ShapeDtypeStruct((R, _COLS), jnp.int32),
            jax.ShapeDtypeStruct((R, _COLS), jnp.float32),
            jax.ShapeDtypeStruct((R, 128), jnp.float32),
        ],
    )(rois_p, tab)
    r_sc = (R * 5 // 8) // 32 * 32
    n_tc = R - r_sc
    idx3 = idx_all.reshape(R, _NCHUNK * 32, 4)[:, :, 0].reshape(R, _NCHUNK, 32)
    w3 = w_all.reshape(R, _NCHUNK, 128)

    rois_per_tile = r_sc // 32
    mesh = plsc.VectorSubcoreMesh(core_axis_name="c", subcore_axis_name="s")
    out_sc = pl.kernel(
        functools.partial(_sc_body, rois_per_tile=rois_per_tile),
        out_type=jax.ShapeDtypeStruct((r_sc, C * _NBIN), jnp.float32),
        mesh=mesh,
        compiler_params=pltpu.CompilerParams(needs_layout_passes=False),
        scratch_types=[
            pltpu.VMEM((_NCHUNK, 32), jnp.int32),
            pltpu.VMEM((_NCHUNK, 128), jnp.float32),
            pltpu.VMEM((2, 32, 512), jnp.int32),
            pltpu.VMEM((_NCHUNK * 8 * 256,), jnp.float32),
            pltpu.VMEM((C * _NBIN,), jnp.float32),
            pltpu.SemaphoreType.DMA,
            pltpu.SemaphoreType.DMA,
        ],
    )(feats, idx3[:r_sc], w3[:r_sc])

    idx_q = idx_all.reshape(R, _NCHUNK * 32, 4)[:, :, 0]
    idxT = idx_q[r_sc:][:, :, None]
    wT = jnp.transpose(w_all[r_sc:].reshape(n_tc, 224, 4), (0, 2, 1))[:, :, :, None]
    scal = scal_all[r_sc:, :8]
    out_tc = pl.pallas_call(
        functools.partial(_tc_roi_body, H=H, W=W, n_tc=n_tc),
        grid=(n_tc,),
        in_specs=[
            pl.BlockSpec(memory_space=pltpu.SMEM),
            pl.BlockSpec((1, 224, 1), lambda i: (i, 0, 0)),
            pl.BlockSpec((1, 4, 224, 1), lambda i: (i, 0, 0, 0)),
            pl.BlockSpec(memory_space=pl.ANY),
        ],
        out_specs=pl.BlockSpec((1, 49, 256), lambda i: (i, 0, 0)),
        out_shape=jax.ShapeDtypeStruct((n_tc, 49, 256), jnp.float32),
        scratch_shapes=[
            pltpu.VMEM((2, 42, 48, 256), jnp.float32),
            pltpu.SemaphoreType.DMA,
            pltpu.SemaphoreType.DMA,
        ],
    )(scal, idxT, wT, feats_nhwc)
    out_tc = jnp.transpose(out_tc, (0, 2, 1))
    out = jnp.concatenate([out_sc.reshape(r_sc, C, _NBIN), out_tc], axis=0)
    return out.reshape(R, C, _OUT_H, _OUT_W)


# speedup vs baseline: 1.0559x; 1.0559x over previous
import functools
import numpy as np
import jax
import jax.numpy as jnp
from jax import lax
from jax.experimental import pallas as pl
from jax.experimental.pallas import tpu as pltpu
from jax.experimental.pallas import tpu_sc as plsc

_OUT_H = 7
_OUT_W = 7
_SCALE = 0.125
_G = 2
_O = 8
_NBIN = _OUT_H * _OUT_W
_NCHUNK = 7
_COLS = _NCHUNK * 128
_PCOL = 880
_RBLK = 64


def _const_table():
    cols = np.arange(_COLS)
    bin_ = cols >> 4
    s = (cols >> 2) & 3
    k = cols & 3
    h = np.minimum(bin_ // _OUT_W, _OUT_H - 1)
    w = bin_ % _OUT_W
    sh = s >> 1
    sw = s & 1
    t = np.zeros((8, _COLS), np.float32)
    t[0] = h
    t[1] = w
    t[2] = (sh + 0.5) / _G
    t[3] = (sw + 0.5) / _G
    t[4] = (k < 2)
    t[5] = (k % 2 == 0)
    t[6] = (bin_ < _NBIN)
    return jnp.asarray(t)


def _prep_body(rois_ref, tab_ref, idx_ref, w_ref, scal_ref, *, H, W):
    r = rois_ref[...]
    b = r[:, 0:1]
    cx = r[:, 1:2] * _SCALE
    cy = r[:, 2:3] * _SCALE
    rw = jnp.maximum(r[:, 3:4] * _SCALE, 1.0)
    rh = jnp.maximum(r[:, 4:5] * _SCALE, 1.0)
    th = r[:, 5:6]
    cos_t = jnp.cos(th)
    sin_t = jnp.sin(th)
    binh = rh / _OUT_H
    binw = rw / _OUT_W
    bh = tab_ref[0:1, :]
    bw = tab_ref[1:2, :]
    sy = tab_ref[2:3, :]
    sx = tab_ref[3:4, :]
    ysel = tab_ref[4:5, :]
    xsel = tab_ref[5:6, :]
    wmask = tab_ref[6:7, :]
    yy = rh * (-0.5) + (bh + sy) * binh
    xx = rw * (-0.5) + (bw + sx) * binw
    y = yy * cos_t - xx * sin_t + cy
    x = yy * sin_t + xx * cos_t + cx
    Hf = float(H)
    Wf = float(W)
    valid = ((y >= -1.0) & (y <= Hf) & (x >= -1.0) & (x <= Wf)).astype(jnp.float32)
    yc = jnp.maximum(y, 0.0)
    yl0 = jnp.floor(yc)
    condy = yl0 >= Hf - 1.0
    y_low = jnp.where(condy, Hf - 1.0, yl0)
    y_high = jnp.where(condy, Hf - 1.0, jnp.minimum(yl0 + 1.0, Hf - 1.0))
    yc = jnp.where(condy, Hf - 1.0, yc)
    ly = yc - y_low
    hy = 1.0 - ly
    xc = jnp.maximum(x, 0.0)
    xl0 = jnp.floor(xc)
    condx = xl0 >= Wf - 1.0
    x_low = jnp.where(condx, Wf - 1.0, xl0)
    x_high = jnp.where(condx, Wf - 1.0, jnp.minimum(xl0 + 1.0, Wf - 1.0))
    xc = jnp.where(condx, Wf - 1.0, xc)
    lx = xc - x_low
    hx = 1.0 - lx
    wy = jnp.where(ysel > 0.0, hy, ly)
    wx = jnp.where(xsel > 0.0, hx, lx)
    wgt = wy * wx * valid * (0.25 * wmask)
    idxf = b * (Hf * Wf) + y_low * Wf + x_low
    indf = th * (_O / (2.0 * np.pi))
    indfl = jnp.floor(indf)
    l_var = indf - indfl
    r_var = 1.0 - l_var
    ind_i = indfl - 8.0 * jnp.floor(indfl * 0.125)
    colid = lax.broadcasted_iota(jnp.int32, wgt.shape, 1)
    w_out = jnp.where(colid == _PCOL, r_var,
                      jnp.where(colid == _PCOL + 1, l_var, wgt))
    idx_out = jnp.where(colid < _NBIN * 16, idxf,
                        jnp.where(colid == _PCOL, ind_i, 0.0))
    idx_ref[...] = idx_out.astype(jnp.int32)
    w_ref[...] = w_out
    yl_m = jnp.where(wmask > 0.0, y_low, Hf)
    xl_m = jnp.where(wmask > 0.0, x_low, Wf)
    y0 = jnp.minimum(jnp.min(yl_m, axis=1, keepdims=True), Hf - 40.0)
    x0 = jnp.minimum(jnp.floor(jnp.min(xl_m, axis=1, keepdims=True) * 0.125) * 8.0,
                     Wf - 48.0)
    c2 = lax.broadcasted_iota(jnp.int32, scal_ref.shape, 1)
    scal = jnp.where(c2 == 0, b,
           jnp.where(c2 == 1, y0,
           jnp.where(c2 == 2, x0,
           jnp.where(c2 == 3, ind_i,
           jnp.where(c2 == 4, r_var,
           jnp.where(c2 == 5, l_var, 0.0))))))
    scal_ref[...] = scal


def _sc_body(feats_hbm, idx_hbm, w_hbm, out_hbm,
             idx_v, w_v, rows_v, pooled_v, out_v, semA, semB,
             *, rois_per_tile):
    cid = lax.axis_index("c")
    sid = lax.axis_index("s")
    wid = sid * 2 + cid

    def start(c, buf, sem):
        pltpu.make_async_copy(feats_hbm.at[idx_v.at[c]], rows_v.at[buf], sem).start()

    def wait(buf, sem):
        pltpu.make_async_copy(feats_hbm.at[idx_v.at[0]], rows_v.at[buf], sem).wait()

    def compute(c, buf):
        def lb_body(lb, _):
            base = lb * 16
            wvec = w_v[c, pl.ds(base, 16)]
            ws = [wvec[k] for k in range(16)]
            binrow = (c * 8 + lb) * 256
            for j2 in range(8):
                accA = None
                accB = None
                for s in range(4):
                    for seg in range(4):
                        wv = rows_v[buf, lb * 4 + s, pl.ds(seg * 128 + 16 * j2, 16)]
                        a = plsc.bitcast(lax.shift_left(wv, 16), jnp.float32)
                        bo = plsc.bitcast(wv & jnp.int32(-65536), jnp.float32)
                        wk = ws[s * 4 + seg]
                        if accA is None:
                            accA = wk * a
                            accB = wk * bo
                        else:
                            accA = accA + wk * a
                            accB = accB + wk * bo
                pooled_v[pl.ds(binrow + 32 * j2, 16)] = accA
                pooled_v[pl.ds(binrow + 32 * j2 + 16, 16)] = accB
            return 0
        lax.fori_loop(0, 8, lb_body, 0)

    def roi_body(i, _):
        roi = wid * rois_per_tile + i
        pltpu.sync_copy(idx_hbm.at[roi], idx_v)
        pltpu.sync_copy(w_hbm.at[roi], w_v)
        pvec_i = idx_v[6, pl.ds(16, 16)]
        pvec_w = w_v[6, pl.ds(112, 16)]
        ind = pvec_i[12]
        rv = pvec_w[0]
        lv = pvec_w[1]
        start(0, 0, semA)

        def pair_body(t, _):
            c0 = 2 * t
            start(c0 + 1, 1, semB)
            wait(0, semA)
            compute(c0, 0)
            start(c0 + 2, 0, semA)
            wait(1, semB)
            compute(c0 + 1, 1)
            return 0
        lax.fori_loop(0, 3, pair_body, 0)
        wait(0, semA)
        compute(6, 0)

        iota = lax.iota(jnp.int32, 16)
        for j in range(16):
            cvec = iota + 16 * j
            grp = cvec & (-8)
            o = cvec & 7
            sA = grp | ((o - ind) & 7)
            sB = grp | ((o - ind + 1) & 7)
            pA = (sA & (-32)) | ((sA & 1) << 4) | ((sA & 31) >> 1)
            pB = (sB & (-32)) | ((sB & 1) << 4) | ((sB & 31) >> 1)
            dstb = cvec * _NBIN

            def blend_body(bn, _):
                a = plsc.load_gather(pooled_v, [pA + bn * 256])
                bb = plsc.load_gather(pooled_v, [pB + bn * 256])
                plsc.store_scatter(out_v, [dstb + bn], rv * a + lv * bb)
                return 0
            lax.fori_loop(0, _NBIN, blend_body, 0)
        pltpu.sync_copy(out_v, out_hbm.at[roi])
        return 0
    lax.fori_loop(0, rois_per_tile, roi_body, 0)


def _tc_roi_body(scal_sm, idxT_ref, wT_ref, feats_any, out_ref,
                 patch_v, semP0, semP1, *, H, W, n_tc):
    i = pl.program_id(0)
    HW = H * W

    def getp(r):
        return (scal_sm[r, 0].astype(jnp.int32),
                scal_sm[r, 1].astype(jnp.int32),
                pl.multiple_of(scal_sm[r, 2].astype(jnp.int32), 8))

    def dma(r, buf, sem):
        b, y0, x0 = getp(r)
        return pltpu.make_async_copy(
            feats_any.at[b, pl.ds(y0, 40), pl.ds(x0, 48), :],
            patch_v.at[buf, pl.ds(0, 40)], sem)

    @pl.when(i == 0)
    def _first():
        patch_v[0, 40:42] = jnp.zeros((2, 48, 256), jnp.float32)
        patch_v[1, 40:42] = jnp.zeros((2, 48, 256), jnp.float32)
        dma(0, 0, semP0).start()

    nxt = jnp.minimum(i + 1, n_tc - 1)

    @pl.when((i + 1 < n_tc) & (i % 2 == 0))
    def _s1():
        dma(nxt, 1, semP1).start()

    @pl.when((i + 1 < n_tc) & (i % 2 == 1))
    def _s0():
        dma(nxt, 0, semP0).start()

    @pl.when(i % 2 == 0)
    def _w0():
        dma(i, 0, semP0).wait()

    @pl.when(i % 2 == 1)
    def _w1():
        dma(i, 1, semP1).wait()

    buf = i % 2
    b, y0, x0 = getp(i)
    pflat = patch_v[buf].reshape(42 * 48, 256)
    pbf = pflat.astype(jnp.bfloat16)
    idxv = idxT_ref[0]
    rel = idxv - b * HW
    yq = lax.shift_right_arithmetic(rel, 7)
    xq = rel & (W - 1)
    cell = (yq - y0) * 48 + (xq - x0)
    ci = lax.broadcasted_iota(jnp.int32, (224, 1920), 1)
    P1 = jnp.where(ci == cell, 1.0, 0.0).astype(jnp.bfloat16)
    res = None
    for k, off in enumerate((0, 1, 48, 49)):
        rk = lax.dot_general(P1, pbf[off:off + 1920, :],
                             (((1,), (0,)), ((), ())),
                             preferred_element_type=jnp.float32)
        wk = wT_ref[0, k]
        res = wk * rk if res is None else res + wk * rk
    bins = res.reshape(56, 4, 256).sum(axis=1)[:49]
    ind = scal_sm[i, 3].astype(jnp.int32)
    rv = scal_sm[i, 4]
    lv = scal_sm[i, 5]
    cp = lax.broadcasted_iota(jnp.int32, (256, 256), 0)
    cc = lax.broadcasted_iota(jnp.int32, (256, 256), 1)
    srcA = (cc & (-8)) | (((cc & 7) - ind) & 7)
    srcB = (cc & (-8)) | (((cc & 7) - ind + 1) & 7)
    B = (jnp.where(cp == srcA, rv, 0.0) + jnp.where(cp == srcB, lv, 0.0))
    outv = lax.dot_general(bins, B, (((1,), (0,)), ((), ())),
                           preferred_element_type=jnp.float32)
    out_ref[0] = outv


def kernel(features, rois):
    N, C, H, W = features.shape
    R = rois.shape[0]
    NHW = N * H * W
    feats_nhwc = jnp.transpose(features, (0, 2, 3, 1))
    fb = feats_nhwc.reshape(NHW, C).astype(jnp.bfloat16)
    fb = jnp.pad(fb, ((0, W + 2), (0, 0)))
    quad = jnp.concatenate(
        [fb[0:NHW], fb[1:NHW + 1], fb[W:NHW + W], fb[W + 1:NHW + W + 1]], axis=1)
    feats = lax.bitcast_convert_type(quad.reshape(NHW, 2 * C, 2), jnp.int32)
    rois_p = jnp.pad(rois, ((0, 0), (0, 128 - rois.shape[1])))
    tab = _const_table()
    idx_all, w_all, scal_all = pl.pallas_call(
        functools.partial(_prep_body, H=H, W=W),
        grid=(R // _RBLK,),
        in_specs=[
            pl.BlockSpec((_RBLK, 128), lambda i: (i, 0)),
            pl.BlockSpec((8, _COLS), lambda i: (0, 0)),
        ],
        out_specs=[
            pl.BlockSpec((_RBLK, _COLS), lambda i: (i, 0)),
            pl.BlockSpec((_RBLK, _COLS), lambda i: (i, 0)),
            pl.BlockSpec((_RBLK, 128), lambda i: (i, 0)),
        ],
        out_shape=[
            jax.ShapeDtypeStruct((R, _COLS), jnp.int32),
            jax.ShapeDtypeStruct((R, _COLS), jnp.float32),
            jax.ShapeDtypeStruct((R, 128), jnp.float32),
        ],
    )(rois_p, tab)
    r_sc = (R * 6 // 8) // 32 * 32
    n_tc = R - r_sc
    idx3 = idx_all.reshape(R, _NCHUNK * 32, 4)[:, :, 0].reshape(R, _NCHUNK, 32)
    w3 = w_all.reshape(R, _NCHUNK, 128)

    rois_per_tile = r_sc // 32
    mesh = plsc.VectorSubcoreMesh(core_axis_name="c", subcore_axis_name="s")
    out_sc = pl.kernel(
        functools.partial(_sc_body, rois_per_tile=rois_per_tile),
        out_type=jax.ShapeDtypeStruct((r_sc, C * _NBIN), jnp.float32),
        mesh=mesh,
        compiler_params=pltpu.CompilerParams(needs_layout_passes=False),
        scratch_types=[
            pltpu.VMEM((_NCHUNK, 32), jnp.int32),
            pltpu.VMEM((_NCHUNK, 128), jnp.float32),
            pltpu.VMEM((2, 32, 512), jnp.int32),
            pltpu.VMEM((_NCHUNK * 8 * 256,), jnp.float32),
            pltpu.VMEM((C * _NBIN,), jnp.float32),
            pltpu.SemaphoreType.DMA,
            pltpu.SemaphoreType.DMA,
        ],
    )(feats, idx3[:r_sc], w3[:r_sc])

    idx_q = idx_all.reshape(R, _NCHUNK * 32, 4)[:, :, 0]
    idxT = idx_q[r_sc:][:, :, None]
    wT = jnp.transpose(w_all[r_sc:].reshape(n_tc, 224, 4), (0, 2, 1))[:, :, :, None]
    scal = scal_all[r_sc:, :8]
    out_tc = pl.pallas_call(
        functools.partial(_tc_roi_body, H=H, W=W, n_tc=n_tc),
        grid=(n_tc,),
        in_specs=[
            pl.BlockSpec(memory_space=pltpu.SMEM),
            pl.BlockSpec((1, 224, 1), lambda i: (i, 0, 0)),
            pl.BlockSpec((1, 4, 224, 1), lambda i: (i, 0, 0, 0)),
            pl.BlockSpec(memory_space=pl.ANY),
        ],
        out_specs=pl.BlockSpec((1, 49, 256), lambda i: (i, 0, 0)),
        out_shape=jax.ShapeDtypeStruct((n_tc, 49, 256), jnp.float32),
        scratch_shapes=[
            pltpu.VMEM((2, 42, 48, 256), jnp.float32),
            pltpu.SemaphoreType.DMA,
            pltpu.SemaphoreType.DMA,
        ],
    )(scal, idxT, wT, feats_nhwc)
    out_tc = jnp.transpose(out_tc, (0, 2, 1))
    out = jnp.concatenate([out_sc.reshape(r_sc, C, _NBIN), out_tc], axis=0)
    return out.reshape(R, C, _OUT_H, _OUT_W)


# speedup vs baseline: 1.0854x; 1.0279x over previous
import functools
import numpy as np
import jax
import jax.numpy as jnp
from jax import lax
from jax.experimental import pallas as pl
from jax.experimental.pallas import tpu as pltpu
from jax.experimental.pallas import tpu_sc as plsc

_OUT_H = 7
_OUT_W = 7
_SCALE = 0.125
_G = 2
_O = 8
_NBIN = _OUT_H * _OUT_W
_NCHUNK = 7
_COLS = _NCHUNK * 128
_PCOL = 880
_RBLK = 64


def _const_table():
    cols = np.arange(_COLS)
    bin_ = cols >> 4
    s = (cols >> 2) & 3
    k = cols & 3
    h = np.minimum(bin_ // _OUT_W, _OUT_H - 1)
    w = bin_ % _OUT_W
    sh = s >> 1
    sw = s & 1
    t = np.zeros((8, _COLS), np.float32)
    t[0] = h
    t[1] = w
    t[2] = (sh + 0.5) / _G
    t[3] = (sw + 0.5) / _G
    t[4] = (k < 2)
    t[5] = (k % 2 == 0)
    t[6] = (bin_ < _NBIN)
    return jnp.asarray(t)


def _prep_body(rois_ref, tab_ref, idx_ref, w_ref, *, H, W):
    r = rois_ref[...]
    b = r[:, 0:1]
    cx = r[:, 1:2] * _SCALE
    cy = r[:, 2:3] * _SCALE
    rw = jnp.maximum(r[:, 3:4] * _SCALE, 1.0)
    rh = jnp.maximum(r[:, 4:5] * _SCALE, 1.0)
    th = r[:, 5:6]
    cos_t = jnp.cos(th)
    sin_t = jnp.sin(th)
    binh = rh / _OUT_H
    binw = rw / _OUT_W
    bh = tab_ref[0:1, :]
    bw = tab_ref[1:2, :]
    sy = tab_ref[2:3, :]
    sx = tab_ref[3:4, :]
    ysel = tab_ref[4:5, :]
    xsel = tab_ref[5:6, :]
    wmask = tab_ref[6:7, :]
    yy = rh * (-0.5) + (bh + sy) * binh
    xx = rw * (-0.5) + (bw + sx) * binw
    y = yy * cos_t - xx * sin_t + cy
    x = yy * sin_t + xx * cos_t + cx
    Hf = float(H)
    Wf = float(W)
    valid = ((y >= -1.0) & (y <= Hf) & (x >= -1.0) & (x <= Wf)).astype(jnp.float32)
    yc = jnp.maximum(y, 0.0)
    yl0 = jnp.floor(yc)
    condy = yl0 >= Hf - 1.0
    y_low = jnp.where(condy, Hf - 1.0, yl0)
    y_high = jnp.where(condy, Hf - 1.0, jnp.minimum(yl0 + 1.0, Hf - 1.0))
    yc = jnp.where(condy, Hf - 1.0, yc)
    ly = yc - y_low
    hy = 1.0 - ly
    xc = jnp.maximum(x, 0.0)
    xl0 = jnp.floor(xc)
    condx = xl0 >= Wf - 1.0
    x_low = jnp.where(condx, Wf - 1.0, xl0)
    x_high = jnp.where(condx, Wf - 1.0, jnp.minimum(xl0 + 1.0, Wf - 1.0))
    xc = jnp.where(condx, Wf - 1.0, xc)
    lx = xc - x_low
    hx = 1.0 - lx
    wy = jnp.where(ysel > 0.0, hy, ly)
    wx = jnp.where(xsel > 0.0, hx, lx)
    wgt = wy * wx * valid * (0.25 * wmask)
    idxf = b * (Hf * Wf) + y_low * Wf + x_low
    indf = th * (_O / (2.0 * np.pi))
    indfl = jnp.floor(indf)
    l_var = indf - indfl
    r_var = 1.0 - l_var
    ind_i = indfl - 8.0 * jnp.floor(indfl * 0.125)
    colid = lax.broadcasted_iota(jnp.int32, wgt.shape, 1)
    w_out = jnp.where(colid == _PCOL, r_var,
                      jnp.where(colid == _PCOL + 1, l_var, wgt))
    idx_out = jnp.where(colid < _NBIN * 16, idxf,
                        jnp.where(colid == _PCOL, ind_i, 0.0))
    idx_ref[...] = idx_out.astype(jnp.int32)
    w_ref[...] = w_out


def _sc_body(feats_hbm, idx_hbm, w_hbm, out_hbm,
             idx_v, w_v, rows_v, pooled_v, out_v, semA, semB,
             *, rois_per_tile):
    cid = lax.axis_index("c")
    sid = lax.axis_index("s")
    wid = sid * 2 + cid

    def start(c, buf, sem):
        pltpu.make_async_copy(feats_hbm.at[idx_v.at[c]], rows_v.at[buf], sem).start()

    def wait(buf, sem):
        pltpu.make_async_copy(feats_hbm.at[idx_v.at[0]], rows_v.at[buf], sem).wait()

    def compute(c, buf):
        def lb_body(lb, _):
            base = lb * 16
            wvec = w_v[c, pl.ds(base, 16)]
            ws = [wvec[k] for k in range(16)]
            binrow = (c * 8 + lb) * 256
            for j2 in range(8):
                accA = None
                accB = None
                for s in range(4):
                    for seg in range(4):
                        wv = rows_v[buf, lb * 4 + s, pl.ds(seg * 128 + 16 * j2, 16)]
                        a = plsc.bitcast(lax.shift_left(wv, 16), jnp.float32)
                        bo = plsc.bitcast(wv & jnp.int32(-65536), jnp.float32)
                        wk = ws[s * 4 + seg]
                        if accA is None:
                            accA = wk * a
                            accB = wk * bo
                        else:
                            accA = accA + wk * a
                            accB = accB + wk * bo
                pooled_v[pl.ds(binrow + 32 * j2, 16)] = accA
                pooled_v[pl.ds(binrow + 32 * j2 + 16, 16)] = accB
            return 0
        lax.fori_loop(0, 8, lb_body, 0)

    def roi_body(i, _):
        roi = wid * rois_per_tile + i
        pltpu.sync_copy(idx_hbm.at[roi], idx_v)
        pltpu.sync_copy(w_hbm.at[roi], w_v)
        pvec_i = idx_v[6, pl.ds(16, 16)]
        pvec_w = w_v[6, pl.ds(112, 16)]
        ind = pvec_i[12]
        rv = pvec_w[0]
        lv = pvec_w[1]
        start(0, 0, semA)

        def pair_body(t, _):
            c0 = 2 * t
            start(c0 + 1, 1, semB)
            wait(0, semA)
            compute(c0, 0)
            start(c0 + 2, 0, semA)
            wait(1, semB)
            compute(c0 + 1, 1)
            return 0
        lax.fori_loop(0, 3, pair_body, 0)
        wait(0, semA)
        compute(6, 0)

        iota = lax.iota(jnp.int32, 16)
        for j in range(16):
            cvec = iota + 16 * j
            grp = cvec & (-8)
            o = cvec & 7
            sA = grp | ((o - ind) & 7)
            sB = grp | ((o - ind + 1) & 7)
            pA = (sA & (-32)) | ((sA & 1) << 4) | ((sA & 31) >> 1)
            pB = (sB & (-32)) | ((sB & 1) << 4) | ((sB & 31) >> 1)
            dstb = cvec * _NBIN

            def blend_body(bn, _):
                a = plsc.load_gather(pooled_v, [pA + bn * 256])
                bb = plsc.load_gather(pooled_v, [pB + bn * 256])
                plsc.store_scatter(out_v, [dstb + bn], rv * a + lv * bb)
                return 0
            lax.fori_loop(0, _NBIN, blend_body, 0)
        pltpu.sync_copy(out_v, out_hbm.at[roi])
        return 0
    lax.fori_loop(0, rois_per_tile, roi_body, 0)


def kernel(features, rois):
    N, C, H, W = features.shape
    R = rois.shape[0]
    NHW = N * H * W
    fb = jnp.transpose(features, (0, 2, 3, 1)).reshape(NHW, C).astype(jnp.bfloat16)
    fb = jnp.pad(fb, ((0, W + 2), (0, 0)))
    quad = jnp.concatenate(
        [fb[0:NHW], fb[1:NHW + 1], fb[W:NHW + W], fb[W + 1:NHW + W + 1]], axis=1)
    feats = lax.bitcast_convert_type(quad.reshape(NHW, 2 * C, 2), jnp.int32)
    rois_p = jnp.pad(rois, ((0, 0), (0, 128 - rois.shape[1])))
    tab = _const_table()
    idx_all, w_all = pl.pallas_call(
        functools.partial(_prep_body, H=H, W=W),
        grid=(R // _RBLK,),
        in_specs=[
            pl.BlockSpec((_RBLK, 128), lambda i: (i, 0)),
            pl.BlockSpec((8, _COLS), lambda i: (0, 0)),
        ],
        out_specs=[
            pl.BlockSpec((_RBLK, _COLS), lambda i: (i, 0)),
            pl.BlockSpec((_RBLK, _COLS), lambda i: (i, 0)),
        ],
        out_shape=[
            jax.ShapeDtypeStruct((R, _COLS), jnp.int32),
            jax.ShapeDtypeStruct((R, _COLS), jnp.float32),
        ],
    )(rois_p, tab)
    idx3 = idx_all.reshape(R, _NCHUNK * 32, 4)[:, :, 0].reshape(R, _NCHUNK, 32)
    w3 = w_all.reshape(R, _NCHUNK, 128)

    rois_per_tile = R // 32
    mesh = plsc.VectorSubcoreMesh(core_axis_name="c", subcore_axis_name="s")
    out = pl.kernel(
        functools.partial(_sc_body, rois_per_tile=rois_per_tile),
        out_type=jax.ShapeDtypeStruct((R, C * _NBIN), jnp.float32),
        mesh=mesh,
        compiler_params=pltpu.CompilerParams(needs_layout_passes=False),
        scratch_types=[
            pltpu.VMEM((_NCHUNK, 32), jnp.int32),
            pltpu.VMEM((_NCHUNK, 128), jnp.float32),
            pltpu.VMEM((2, 32, 512), jnp.int32),
            pltpu.VMEM((_NCHUNK * 8 * 256,), jnp.float32),
            pltpu.VMEM((C * _NBIN,), jnp.float32),
            pltpu.SemaphoreType.DMA,
            pltpu.SemaphoreType.DMA,
        ],
    )(feats, idx3, w3)
    return out.reshape(R, C, _OUT_H, _OUT_W)


# speedup vs baseline: 1.0867x; 1.0012x over previous
import functools
import numpy as np
import jax
import jax.numpy as jnp
from jax import lax
from jax.experimental import pallas as pl
from jax.experimental.pallas import tpu as pltpu
from jax.experimental.pallas import tpu_sc as plsc

_OUT_H = 7
_OUT_W = 7
_SCALE = 0.125
_G = 2
_O = 8
_NBIN = _OUT_H * _OUT_W
_NCHUNK = 7
_COLS = _NCHUNK * 128
_PCOL = 880
_RBLK = 64


def _const_table():
    cols = np.arange(_COLS)
    bin_ = cols >> 4
    s = (cols >> 2) & 3
    k = cols & 3
    h = np.minimum(bin_ // _OUT_W, _OUT_H - 1)
    w = bin_ % _OUT_W
    sh = s >> 1
    sw = s & 1
    t = np.zeros((8, _COLS), np.float32)
    t[0] = h
    t[1] = w
    t[2] = (sh + 0.5) / _G
    t[3] = (sw + 0.5) / _G
    t[4] = (k < 2)
    t[5] = (k % 2 == 0)
    t[6] = (bin_ < _NBIN)
    return jnp.asarray(t)


def _prep_body(rois_ref, tab_ref, idx_ref, w_ref, *, H, W):
    r = rois_ref[...]
    b = r[:, 0:1]
    cx = r[:, 1:2] * _SCALE
    cy = r[:, 2:3] * _SCALE
    rw = jnp.maximum(r[:, 3:4] * _SCALE, 1.0)
    rh = jnp.maximum(r[:, 4:5] * _SCALE, 1.0)
    th = r[:, 5:6]
    cos_t = jnp.cos(th)
    sin_t = jnp.sin(th)
    binh = rh / _OUT_H
    binw = rw / _OUT_W
    bh = tab_ref[0:1, :]
    bw = tab_ref[1:2, :]
    sy = tab_ref[2:3, :]
    sx = tab_ref[3:4, :]
    ysel = tab_ref[4:5, :]
    xsel = tab_ref[5:6, :]
    wmask = tab_ref[6:7, :]
    yy = rh * (-0.5) + (bh + sy) * binh
    xx = rw * (-0.5) + (bw + sx) * binw
    y = yy * cos_t - xx * sin_t + cy
    x = yy * sin_t + xx * cos_t + cx
    Hf = float(H)
    Wf = float(W)
    valid = ((y >= -1.0) & (y <= Hf) & (x >= -1.0) & (x <= Wf)).astype(jnp.float32)
    yc = jnp.maximum(y, 0.0)
    yl0 = jnp.floor(yc)
    condy = yl0 >= Hf - 1.0
    y_low = jnp.where(condy, Hf - 1.0, yl0)
    y_high = jnp.where(condy, Hf - 1.0, jnp.minimum(yl0 + 1.0, Hf - 1.0))
    yc = jnp.where(condy, Hf - 1.0, yc)
    ly = yc - y_low
    hy = 1.0 - ly
    xc = jnp.maximum(x, 0.0)
    xl0 = jnp.floor(xc)
    condx = xl0 >= Wf - 1.0
    x_low = jnp.where(condx, Wf - 1.0, xl0)
    x_high = jnp.where(condx, Wf - 1.0, jnp.minimum(xl0 + 1.0, Wf - 1.0))
    xc = jnp.where(condx, Wf - 1.0, xc)
    lx = xc - x_low
    hx = 1.0 - lx
    wy = jnp.where(ysel > 0.0, hy, ly)
    wx = jnp.where(xsel > 0.0, hx, lx)
    wgt = wy * wx * valid * (0.25 * wmask)
    idxf = b * (Hf * Wf) + y_low * Wf + x_low
    indf = th * (_O / (2.0 * np.pi))
    indfl = jnp.floor(indf)
    l_var = indf - indfl
    r_var = 1.0 - l_var
    ind_i = indfl - 8.0 * jnp.floor(indfl * 0.125)
    colid = lax.broadcasted_iota(jnp.int32, wgt.shape, 1)
    w_out = jnp.where(colid == _PCOL, r_var,
                      jnp.where(colid == _PCOL + 1, l_var, wgt))
    idx_out = jnp.where(colid < _NBIN * 16, idxf,
                        jnp.where(colid == _PCOL, ind_i, 0.0))
    idx_ref[...] = idx_out.astype(jnp.int32)
    w_ref[...] = w_out


def _sc_body(feats_hbm, idx_hbm, w_hbm, out_hbm,
             idx_v, w_v, rows_v, pooled_v, out_v, semA, semB,
             *, rois_per_tile):
    cid = lax.axis_index("c")
    sid = lax.axis_index("s")
    wid = sid * 2 + cid

    def start(c, buf, sem):
        pltpu.make_async_copy(feats_hbm.at[idx_v.at[c]], rows_v.at[buf],
                              sem).start()

    def wait(buf, sem):
        pltpu.make_async_copy(feats_hbm.at[idx_v.at[0]], rows_v.at[buf],
                              sem).wait()

    def compute(c, buf):
        def lb_body(lb, _):
            base = lb * 16
            wvec = w_v[c, pl.ds(base, 16)]
            ws = [wvec[k] for k in range(16)]
            binrow = (c * 8 + lb) * 256
            for j2 in range(8):
                accA = None
                accB = None
                for s in range(4):
                    for seg in range(4):
                        wv = rows_v[buf, lb * 4 + s, pl.ds(seg * 128 + 16 * j2, 16)]
                        a = plsc.bitcast(lax.shift_left(wv, 16), jnp.float32)
                        bo = plsc.bitcast(wv & jnp.int32(-65536), jnp.float32)
                        wk = ws[s * 4 + seg]
                        if accA is None:
                            accA = wk * a
                            accB = wk * bo
                        else:
                            accA = accA + wk * a
                            accB = accB + wk * bo
                pooled_v[pl.ds(binrow + 32 * j2, 16)] = accA
                pooled_v[pl.ds(binrow + 32 * j2 + 16, 16)] = accB
            return 0
        lax.fori_loop(0, 8, lb_body, 0)

    first_roi = wid * rois_per_tile
    pltpu.sync_copy(idx_hbm.at[first_roi], idx_v)
    pltpu.sync_copy(w_hbm.at[first_roi], w_v)
    start(0, 0, semA)

    def roi_body(i, _):
        roi = wid * rois_per_tile + i
        pvec_i = idx_v[6, pl.ds(16, 16)]
        pvec_w = w_v[6, pl.ds(112, 16)]
        ind = pvec_i[12]
        rv = pvec_w[0]
        lv = pvec_w[1]

        def pair_body(t, _):
            c0 = 2 * t
            start(c0 + 1, 1, semB)
            wait(0, semA)
            compute(c0, 0)
            start(c0 + 2, 0, semA)
            wait(1, semB)
            compute(c0 + 1, 1)
            return 0
        lax.fori_loop(0, 3, pair_body, 0)
        wait(0, semA)
        compute(6, 0)

        @pl.when(i + 1 < rois_per_tile)
        def _nx():
            pltpu.sync_copy(idx_hbm.at[roi + 1], idx_v)
            pltpu.sync_copy(w_hbm.at[roi + 1], w_v)
            start(0, 0, semA)

        iota = lax.iota(jnp.int32, 16)
        for j in range(16):
            cvec = iota + 16 * j
            grp = cvec & (-8)
            o = cvec & 7
            sA = grp | ((o - ind) & 7)
            sB = grp | ((o - ind + 1) & 7)
            pA = (sA & (-32)) | ((sA & 1) << 4) | ((sA & 31) >> 1)
            pB = (sB & (-32)) | ((sB & 1) << 4) | ((sB & 31) >> 1)
            dstb = cvec * _NBIN

            def blend_body(bn, _):
                a = plsc.load_gather(pooled_v, [pA + bn * 256])
                bb = plsc.load_gather(pooled_v, [pB + bn * 256])
                plsc.store_scatter(out_v, [dstb + bn], rv * a + lv * bb)
                return 0
            lax.fori_loop(0, _NBIN, blend_body, 0)
        pltpu.sync_copy(out_v, out_hbm.at[roi])
        return 0
    lax.fori_loop(0, rois_per_tile, roi_body, 0)


def kernel(features, rois):
    N, C, H, W = features.shape
    R = rois.shape[0]
    NHW = N * H * W
    fb = jnp.transpose(features, (0, 2, 3, 1)).reshape(NHW, C).astype(jnp.bfloat16)
    fb = jnp.pad(fb, ((0, W + 2), (0, 0)))
    quad = jnp.concatenate(
        [fb[0:NHW], fb[1:NHW + 1], fb[W:NHW + W], fb[W + 1:NHW + W + 1]], axis=1)
    feats = lax.bitcast_convert_type(quad.reshape(NHW, 2 * C, 2), jnp.int32)
    rois_p = jnp.pad(rois, ((0, 0), (0, 128 - rois.shape[1])))
    tab = _const_table()
    idx_all, w_all = pl.pallas_call(
        functools.partial(_prep_body, H=H, W=W),
        grid=(R // _RBLK,),
        in_specs=[
            pl.BlockSpec((_RBLK, 128), lambda i: (i, 0)),
            pl.BlockSpec((8, _COLS), lambda i: (0, 0)),
        ],
        out_specs=[
            pl.BlockSpec((_RBLK, _COLS), lambda i: (i, 0)),
            pl.BlockSpec((_RBLK, _COLS), lambda i: (i, 0)),
        ],
        out_shape=[
            jax.ShapeDtypeStruct((R, _COLS), jnp.int32),
            jax.ShapeDtypeStruct((R, _COLS), jnp.float32),
        ],
    )(rois_p, tab)
    idx3 = idx_all.reshape(R, _NCHUNK * 32, 4)[:, :, 0].reshape(R, _NCHUNK, 32)
    w3 = w_all.reshape(R, _NCHUNK, 128)

    rois_per_tile = R // 32
    mesh = plsc.VectorSubcoreMesh(core_axis_name="c", subcore_axis_name="s")
    out = pl.kernel(
        functools.partial(_sc_body, rois_per_tile=rois_per_tile),
        out_type=jax.ShapeDtypeStruct((R, C * _NBIN), jnp.float32),
        mesh=mesh,
        compiler_params=pltpu.CompilerParams(needs_layout_passes=False),
        scratch_types=[
            pltpu.VMEM((_NCHUNK, 32), jnp.int32),
            pltpu.VMEM((_NCHUNK, 128), jnp.float32),
            pltpu.VMEM((2, 32, 512), jnp.int32),
            pltpu.VMEM((_NCHUNK * 8 * 256,), jnp.float32),
            pltpu.VMEM((C * _NBIN,), jnp.float32),
            pltpu.SemaphoreType.DMA,
            pltpu.SemaphoreType.DMA,
        ],
    )(feats, idx3, w3)
    return out.reshape(R, C, _OUT_H, _OUT_W)
